# Initial kernel scaffold; baseline (speedup 1.0000x reference)
import jax
import jax.numpy as jnp
from jax.experimental import pallas as pl


def kernel(input, V, E, weight, bias):
    raise NotImplementedError("write your pallas kernel here")


# speedup vs baseline: 4.8362x; 4.8362x over previous
import functools

import jax
import jax.numpy as jnp
from jax import lax
from jax.experimental import pallas as pl
from jax.experimental.pallas import tpu as pltpu
from jax.experimental.pallas import tpu_sc as plsc

N_NODE = 10000
N_EDGE = 2000
NNZ = 320000
D = 128
W = 144
ONES_COL = 128
EDGE_PAD = 2048
NODE_PAD = 10240
BLK = 1000
FBLK = 1024
B = 80


def _mm_body(x_ref, w_ref, o_ref):
    acc = jnp.dot(x_ref[...], w_ref[...], preferred_element_type=jnp.float32)
    col = lax.broadcasted_iota(jnp.int32, (BLK, W), 1)
    o_ref[...] = acc + jnp.where(col == ONES_COL, 1.0, 0.0)


def _mm(x, wp):
    return pl.pallas_call(
        _mm_body,
        grid=(N_NODE // BLK,),
        in_specs=[
            pl.BlockSpec((BLK, D), lambda i: (i, 0)),
            pl.BlockSpec((D, W), lambda i: (0, 0)),
        ],
        out_specs=pl.BlockSpec((BLK, W), lambda i: (i, 0)),
        out_shape=jax.ShapeDtypeStruct((N_NODE, W), jnp.float32),
    )(x, wp)


def _merge_body(a_ref, b_ref, o_ref):
    t = a_ref[...] + b_ref[...]
    cnt = jnp.maximum(t[:, ONES_COL:ONES_COL + 1], 1.0)
    o_ref[...] = t / cnt


def _merge(ep):
    return pl.pallas_call(
        _merge_body,
        grid=(1,),
        in_specs=[
            pl.BlockSpec((EDGE_PAD, W), lambda i: (0, 0)),
            pl.BlockSpec((EDGE_PAD, W), lambda i: (1, 0)),
        ],
        out_specs=pl.BlockSpec((EDGE_PAD, W), lambda i: (0, 0)),
        out_shape=jax.ShapeDtypeStruct((EDGE_PAD, W), jnp.float32),
    )(ep, ep)


def _final_body(a_ref, b_ref, bias_ref, o_ref):
    t = a_ref[...] + b_ref[...]
    cnt = jnp.maximum(t[:, ONES_COL:ONES_COL + 1], 1.0)
    o_ref[...] = t[:, :D] / cnt + bias_ref[...]


def _final(vp, bias2d):
    nb = NODE_PAD // FBLK
    return pl.pallas_call(
        _final_body,
        grid=(nb,),
        in_specs=[
            pl.BlockSpec((FBLK, W), lambda i: (i, 0)),
            pl.BlockSpec((FBLK, W), lambda i: (i + nb, 0)),
            pl.BlockSpec((1, D), lambda i: (0, 0)),
        ],
        out_specs=pl.BlockSpec((FBLK, D), lambda i: (i, 0)),
        out_shape=jax.ShapeDtypeStruct((NODE_PAD, D), jnp.float32),
    )(vp, vp, bias2d)


def _make_sc_phase(n_dst):
    rows_per_tile = n_dst // 16
    n_cp = rows_per_tile // 128
    chunk = NNZ // 32
    nb = chunk // B
    mesh = plsc.VectorSubcoreMesh(core_axis_name="c", subcore_axis_name="s")

    @functools.partial(
        pl.kernel,
        mesh=mesh,
        compiler_params=pltpu.CompilerParams(use_tc_tiling_on_sc=False),
        out_type=jax.ShapeDtypeStruct((2 * n_dst, W), jnp.float32),
        scratch_types=[
            pltpu.VMEM((B,), jnp.int32),
            pltpu.VMEM((B,), jnp.int32),
            pltpu.VMEM((B, W), jnp.float32),
            pltpu.VMEM((128, W), jnp.float32),
            pltpu.VMEM_SHARED((n_dst, W), jnp.float32),
            pltpu.SemaphoreType.DMA,
        ],
    )
    def phase(src_hbm, gidx_hbm, didx_hbm, z_hbm, out_hbm,
              gbuf, dbuf, rows, cpbuf, acc, sem):
        c = lax.axis_index("c")
        s = lax.axis_index("s")
        r0 = s * rows_per_tile
        pltpu.sync_copy(z_hbm.at[pl.ds(r0, rows_per_tile)],
                        acc.at[pl.ds(r0, rows_per_tile)])
        plsc.subcore_barrier()
        base = c * (16 * chunk) + s * chunk

        def body(j, carry):
            off = base + j * B
            pltpu.sync_copy(gidx_hbm.at[pl.ds(off, B)], gbuf)
            pltpu.sync_copy(didx_hbm.at[pl.ds(off, B)], dbuf)
            pltpu.async_copy(src_hbm.at[gbuf], rows, sem).wait()
            pltpu.sync_copy(rows, acc.at[dbuf], add=True)
            return carry

        lax.fori_loop(0, nb, body, 0)
        plsc.subcore_barrier()
        for k in range(n_cp):
            q0 = r0 + k * 128
            pltpu.sync_copy(acc.at[pl.ds(q0, 128)], cpbuf)
            pltpu.sync_copy(cpbuf, out_hbm.at[pl.ds(c * n_dst + q0, 128)])

    return phase


_phase_e = _make_sc_phase(EDGE_PAD)
_phase_v = _make_sc_phase(NODE_PAD)


def kernel(input, V, E, weight, bias):
    x = input.astype(jnp.float32)
    v32 = V.astype(jnp.int32)
    e32 = E.astype(jnp.int32)
    wp = jnp.pad(weight.astype(jnp.float32), ((0, 0), (0, W - D)))
    z = jnp.zeros((NODE_PAD, W), jnp.float32)
    xp = _mm(x, wp)
    ep = _phase_e(xp, v32, e32, z)
    xe = _merge(ep)
    vp = _phase_v(xe, e32, v32, z)
    out = _final(vp, bias.reshape(1, D).astype(jnp.float32))
    return out[:N_NODE]


# speedup vs baseline: 8.6259x; 1.7836x over previous
import functools

import jax
import jax.numpy as jnp
from jax import lax
from jax.experimental import pallas as pl
from jax.experimental.pallas import tpu as pltpu
from jax.experimental.pallas import tpu_sc as plsc

N_NODE = 10000
N_EDGE = 2000
NNZ = 320000
D = 128
W = 144
ONES_COL = 128
EDGE_PAD = 2048
NODE_PAD = 10240
BLK = 1000
FBLK = 1024


def _mm_body(x_ref, w_ref, o_ref):
    acc = jnp.dot(x_ref[...], w_ref[...], preferred_element_type=jnp.float32)
    col = lax.broadcasted_iota(jnp.int32, (BLK, W), 1)
    o_ref[...] = acc + jnp.where(col == ONES_COL, 1.0, 0.0)


def _mm(x, wp):
    return pl.pallas_call(
        _mm_body,
        grid=(N_NODE // BLK,),
        in_specs=[
            pl.BlockSpec((BLK, D), lambda i: (i, 0)),
            pl.BlockSpec((D, W), lambda i: (0, 0)),
        ],
        out_specs=pl.BlockSpec((BLK, W), lambda i: (i, 0)),
        out_shape=jax.ShapeDtypeStruct((N_NODE, W), jnp.float32),
    )(x, wp)


def _merge_body(a_ref, b_ref, o_ref):
    t = a_ref[...] + b_ref[...]
    cnt = jnp.maximum(t[:, ONES_COL:ONES_COL + 1], 1.0)
    o_ref[...] = t / cnt


def _merge(ep):
    return pl.pallas_call(
        _merge_body,
        grid=(1,),
        in_specs=[
            pl.BlockSpec((EDGE_PAD, W), lambda i: (0, 0)),
            pl.BlockSpec((EDGE_PAD, W), lambda i: (1, 0)),
        ],
        out_specs=pl.BlockSpec((EDGE_PAD, W), lambda i: (0, 0)),
        out_shape=jax.ShapeDtypeStruct((EDGE_PAD, W), jnp.float32),
    )(ep, ep)


def _final_body(a_ref, b_ref, bias_ref, o_ref):
    t = a_ref[...] + b_ref[...]
    cnt = jnp.maximum(t[:, ONES_COL:ONES_COL + 1], 1.0)
    o_ref[...] = t[:, :D] / cnt + bias_ref[...]


def _final(vp, bias2d):
    nb = NODE_PAD // FBLK
    return pl.pallas_call(
        _final_body,
        grid=(nb,),
        in_specs=[
            pl.BlockSpec((FBLK, W), lambda i: (i, 0)),
            pl.BlockSpec((FBLK, W), lambda i: (i + nb, 0)),
            pl.BlockSpec((1, D), lambda i: (0, 0)),
        ],
        out_specs=pl.BlockSpec((FBLK, D), lambda i: (i, 0)),
        out_shape=jax.ShapeDtypeStruct((NODE_PAD, D), jnp.float32),
    )(vp, vp, bias2d)


def _make_sc_phase(n_dst, B, G):
    rows_per_tile = n_dst // 16
    chunk = NNZ // 32
    nb = chunk // B
    ng = nb // G
    assert nb % G == 0
    mesh = plsc.VectorSubcoreMesh(core_axis_name="c", subcore_axis_name="s")

    @functools.partial(
        pl.kernel,
        mesh=mesh,
        compiler_params=pltpu.CompilerParams(use_tc_tiling_on_sc=False),
        out_type=jax.ShapeDtypeStruct((2 * n_dst, W), jnp.float32),
        scratch_types=[
            pltpu.VMEM((G, B), jnp.int32),
            pltpu.VMEM((2, G, B), jnp.int32),
            pltpu.VMEM_SHARED((n_dst, W), jnp.float32),
        ]
        + [pltpu.VMEM((B, W), jnp.float32) for _ in range(G)]
        + [pltpu.SemaphoreType.DMA for _ in range(2 * G)],
    )
    def phase(src_hbm, gidx_hbm, didx_hbm, z_hbm, out_hbm,
              gibuf, dibuf, acc, *rest):
        rows = rest[:G]
        gsem = rest[G:2 * G]
        ssem = rest[2 * G:]
        c = lax.axis_index("c")
        s = lax.axis_index("s")
        r0 = s * rows_per_tile
        pltpu.sync_copy(z_hbm.at[pl.ds(r0, rows_per_tile)],
                        acc.at[pl.ds(r0, rows_per_tile)])
        tb = (c * 16 + s) * nb
        pltpu.sync_copy(gidx_hbm.at[pl.ds(tb, G)], gibuf)
        pltpu.sync_copy(didx_hbm.at[pl.ds(tb, G)], dibuf.at[0])
        plsc.subcore_barrier()

        def group(g, carry):
            p = lax.rem(g, 2)
            gd = [pltpu.async_copy(src_hbm.at[gibuf.at[b]],
                                   rows[b], gsem[b]) for b in range(G)]
            sd = []
            for b in range(G):
                gd[b].wait()
                sd.append(pltpu.async_copy(rows[b], acc.at[dibuf.at[p, b]],
                                           ssem[b], add=True))
            nxt = tb + jnp.minimum((g + 1) * G, nb - G)
            pltpu.sync_copy(gidx_hbm.at[pl.ds(nxt, G)], gibuf)
            pltpu.sync_copy(didx_hbm.at[pl.ds(nxt, G)], dibuf.at[1 - p])
            for d in sd:
                d.wait()
            return carry

        lax.fori_loop(0, ng, group, 0)
        plsc.subcore_barrier()
        pltpu.sync_copy(acc.at[pl.ds(r0, rows_per_tile)],
                        out_hbm.at[pl.ds(c * n_dst + r0, rows_per_tile)])

    return phase


_phase_e = _make_sc_phase(EDGE_PAD, 80, 5)
_phase_v = _make_sc_phase(NODE_PAD, 40, 5)


def kernel(input, V, E, weight, bias):
    x = input.astype(jnp.float32)
    v32 = V.astype(jnp.int32)
    e32 = E.astype(jnp.int32)
    wp = jnp.pad(weight.astype(jnp.float32), ((0, 0), (0, W - D)))
    z = jnp.zeros((NODE_PAD, W), jnp.float32)
    xp = _mm(x, wp)
    ep = _phase_e(xp, v32.reshape(NNZ // 80, 80), e32.reshape(NNZ // 80, 80),
                  z)
    xe = _merge(ep)
    vp = _phase_v(xe, e32.reshape(NNZ // 40, 40), v32.reshape(NNZ // 40, 40),
                  z)
    out = _final(vp, bias.reshape(1, D).astype(jnp.float32))
    return out[:N_NODE]


# speedup vs baseline: 8.6303x; 1.0005x over previous
import functools

import jax
import jax.numpy as jnp
from jax import lax
from jax.experimental import pallas as pl
from jax.experimental.pallas import tpu as pltpu
from jax.experimental.pallas import tpu_sc as plsc

N_NODE = 10000
N_EDGE = 2000
NNZ = 320000
D = 128
W = 144
ONES_COL = 128
EDGE_PAD = 2048
NODE_PAD = 10240
BLK = 1000
FBLK = 1024


def _mm_body(x_ref, w_ref, o_ref):
    acc = jnp.dot(x_ref[...], w_ref[...], preferred_element_type=jnp.float32)
    col = lax.broadcasted_iota(jnp.int32, (BLK, W), 1)
    o_ref[...] = acc + jnp.where(col == ONES_COL, 1.0, 0.0)


def _mm(x, wp):
    return pl.pallas_call(
        _mm_body,
        grid=(N_NODE // BLK,),
        in_specs=[
            pl.BlockSpec((BLK, D), lambda i: (i, 0)),
            pl.BlockSpec((D, W), lambda i: (0, 0)),
        ],
        out_specs=pl.BlockSpec((BLK, W), lambda i: (i, 0)),
        out_shape=jax.ShapeDtypeStruct((N_NODE, W), jnp.float32),
    )(x, wp)


def _merge_body(a_ref, b_ref, o_ref):
    t = a_ref[...] + b_ref[...]
    cnt = jnp.maximum(t[:, ONES_COL:ONES_COL + 1], 1.0)
    o_ref[...] = t / cnt


def _merge(ep):
    return pl.pallas_call(
        _merge_body,
        grid=(1,),
        in_specs=[
            pl.BlockSpec((EDGE_PAD, W), lambda i: (0, 0)),
            pl.BlockSpec((EDGE_PAD, W), lambda i: (1, 0)),
        ],
        out_specs=pl.BlockSpec((EDGE_PAD, W), lambda i: (0, 0)),
        out_shape=jax.ShapeDtypeStruct((EDGE_PAD, W), jnp.float32),
    )(ep, ep)


def _final_body(a_ref, b_ref, bias_ref, o_ref):
    t = a_ref[...] + b_ref[...]
    cnt = jnp.maximum(t[:, ONES_COL:ONES_COL + 1], 1.0)
    o_ref[...] = t[:, :D] / cnt + bias_ref[...]


def _final(vp, bias2d):
    nb = NODE_PAD // FBLK
    return pl.pallas_call(
        _final_body,
        grid=(nb,),
        in_specs=[
            pl.BlockSpec((FBLK, W), lambda i: (i, 0)),
            pl.BlockSpec((FBLK, W), lambda i: (i + nb, 0)),
            pl.BlockSpec((1, D), lambda i: (0, 0)),
        ],
        out_specs=pl.BlockSpec((FBLK, D), lambda i: (i, 0)),
        out_shape=jax.ShapeDtypeStruct((NODE_PAD, D), jnp.float32),
    )(vp, vp, bias2d)


def _make_sc_phase(n_dst, B, G):
    rows_per_tile = n_dst // 16
    chunk = NNZ // 32
    nb = chunk // B
    ng = nb // G
    assert nb % G == 0
    mesh = plsc.VectorSubcoreMesh(core_axis_name="c", subcore_axis_name="s")

    @functools.partial(
        pl.kernel,
        mesh=mesh,
        compiler_params=pltpu.CompilerParams(use_tc_tiling_on_sc=False),
        out_type=jax.ShapeDtypeStruct((2 * n_dst, W), jnp.float32),
        scratch_types=[
            pltpu.VMEM((G, B), jnp.int32),
            pltpu.VMEM((2, G, B), jnp.int32),
            pltpu.VMEM_SHARED((n_dst, W), jnp.float32),
        ]
        + [pltpu.VMEM((B, W), jnp.float32) for _ in range(G)]
        + [pltpu.SemaphoreType.DMA for _ in range(2 * G)],
    )
    def phase(src_hbm, gidx_hbm, didx_hbm, z_hbm, out_hbm,
              gibuf, dibuf, acc, *rest):
        rows = rest[:G]
        gsem = rest[G:2 * G]
        ssem = rest[2 * G:]
        c = lax.axis_index("c")
        s = lax.axis_index("s")
        r0 = s * rows_per_tile
        pltpu.sync_copy(z_hbm.at[pl.ds(r0, rows_per_tile)],
                        acc.at[pl.ds(r0, rows_per_tile)])
        tb = (c * 16 + s) * nb
        pltpu.sync_copy(gidx_hbm.at[pl.ds(tb, G)], gibuf)
        pltpu.sync_copy(didx_hbm.at[pl.ds(tb, G)], dibuf.at[0])
        plsc.subcore_barrier()

        def group(g, carry):
            p = lax.rem(g, 2)
            gd = [pltpu.async_copy(src_hbm.at[gibuf.at[b]],
                                   rows[b], gsem[b]) for b in range(G)]
            sd = []
            for b in range(G):
                gd[b].wait()
                sd.append(pltpu.async_copy(rows[b], acc.at[dibuf.at[p, b]],
                                           ssem[b], add=True))
            nxt = tb + jnp.minimum((g + 1) * G, nb - G)
            pltpu.sync_copy(gidx_hbm.at[pl.ds(nxt, G)], gibuf)
            pltpu.sync_copy(didx_hbm.at[pl.ds(nxt, G)], dibuf.at[1 - p])
            for d in sd:
                d.wait()
            return carry

        lax.fori_loop(0, ng, group, 0)
        plsc.subcore_barrier()
        pltpu.sync_copy(acc.at[pl.ds(r0, rows_per_tile)],
                        out_hbm.at[pl.ds(c * n_dst + r0, rows_per_tile)])

    return phase


_phase_e = _make_sc_phase(EDGE_PAD, 125, 5)
_phase_v = _make_sc_phase(NODE_PAD, 50, 5)


def kernel(input, V, E, weight, bias):
    x = input.astype(jnp.float32)
    v32 = V.astype(jnp.int32)
    e32 = E.astype(jnp.int32)
    wp = jnp.pad(weight.astype(jnp.float32), ((0, 0), (0, W - D)))
    z = jnp.zeros((NODE_PAD, W), jnp.float32)
    xp = _mm(x, wp)
    ep = _phase_e(xp, v32.reshape(NNZ // 125, 125), e32.reshape(NNZ // 125, 125),
                  z)
    xe = _merge(ep)
    vp = _phase_v(xe, e32.reshape(NNZ // 50, 50), v32.reshape(NNZ // 50, 50),
                  z)
    out = _final(vp, bias.reshape(1, D).astype(jnp.float32))
    return out[:N_NODE]
